# Initial kernel scaffold; baseline (speedup 1.0000x reference)
import jax
import jax.numpy as jnp
from jax.experimental import pallas as pl


def kernel(input_bitarrays, output_node_input_indices, nor_mask):
    raise NotImplementedError("write your pallas kernel here")


# speedup vs baseline: 1.5139x; 1.5139x over previous
import functools

import jax
import jax.numpy as jnp
from jax import lax
from jax.experimental import pallas as pl
from jax.experimental.pallas import tpu as pltpu
from jax.experimental.pallas import tpu_sc as plsc

N_IN = 100000
N_OUT = 100000
W = 128
LANES = 16
NW = 32
ROWS_PER_TILE = N_OUT // NW
CHUNK = 125
CHUNK_PAD = 128
NCHUNK = ROWS_PER_TILE // CHUNK
NROW = N_OUT // CHUNK


def _body(table, idx0, idx1, nmask, out,
          idx0_v, idx1_v, mask_v,
          buf_a0, buf_a1, buf_b0, buf_b1, ob0, ob1,
          insem0, insem1, outsem0, outsem1):
    wid = lax.axis_index("s") * 2 + lax.axis_index("c")
    tile_row = wid * NCHUNK

    pltpu.sync_copy(idx0.at[wid], idx0_v)
    pltpu.sync_copy(idx1.at[wid], idx1_v)
    pltpu.sync_copy(nmask.at[wid], mask_v)

    buf_a = (buf_a0, buf_a1)
    buf_b = (buf_b0, buf_b1)
    ob = (ob0, ob1)
    insem = (insem0, insem1)
    outsem = (outsem0, outsem1)

    def start_gather(ci, s):
        pltpu.async_copy(table.at[idx0_v.at[ci]], buf_a[s], insem[s])
        pltpu.async_copy(table.at[idx1_v.at[ci]], buf_b[s], insem[s])

    def wait_gather(ci, s):
        pltpu.make_async_copy(table.at[idx0_v.at[ci]], buf_a[s], insem[s]).wait()
        pltpu.make_async_copy(table.at[idx1_v.at[ci]], buf_b[s], insem[s]).wait()

    def out_slice(ci):
        return out.at[tile_row + ci]

    def start_out(ci, s):
        pltpu.async_copy(ob[s].at[pl.ds(0, CHUNK)], out_slice(ci), outsem[s])

    def wait_out(ci, s):
        pltpu.make_async_copy(ob[s].at[pl.ds(0, CHUNK)], out_slice(ci),
                              outsem[s]).wait()

    def compute(ci, s):
        a_ref, b_ref, o_ref = buf_a[s], buf_b[s], ob[s]

        def g_body(g, carry):
            base = g * LANES
            m16 = mask_v[ci, pl.ds(base, LANES)]
            for l in range(LANES):
                m = jnp.full((LANES,), m16[l], jnp.int32)
                for w in range(W // LANES):
                    a = a_ref[base + l, pl.ds(w * LANES, LANES)]
                    b = b_ref[base + l, pl.ds(w * LANES, LANES)]
                    o_ref[base + l, pl.ds(w * LANES, LANES)] = (
                        ~((a & b) ^ (m & (a ^ b))))
            return carry

        lax.fori_loop(0, CHUNK_PAD // LANES, g_body, 0)

    start_gather(0, 0)

    def pair_body(p, carry):
        for b in (0, 1):
            ci = 2 * p + b

            @pl.when(ci < NCHUNK)
            def _(ci=ci, b=b):
                @pl.when(ci + 1 < NCHUNK)
                def _():
                    start_gather(ci + 1, 1 - b)

                wait_gather(ci, b)

                @pl.when(ci >= 2)
                def _():
                    wait_out(ci - 2, b)

                compute(ci, b)
                start_out(ci, b)
        return carry

    lax.fori_loop(0, (NCHUNK + 1) // 2, pair_body, 0)
    wait_out(NCHUNK - 2, (NCHUNK - 2) % 2)
    wait_out(NCHUNK - 1, (NCHUNK - 1) % 2)


@jax.jit
def _nand_layer(table, idx0, idx1, nmask):
    mesh = plsc.VectorSubcoreMesh(core_axis_name="c", subcore_axis_name="s")
    f = functools.partial(
        pl.kernel,
        out_type=jax.ShapeDtypeStruct((NROW, CHUNK, W), jnp.int32),
        mesh=mesh,
        scratch_types=[
            pltpu.VMEM((NCHUNK, CHUNK_PAD), jnp.int32),
            pltpu.VMEM((NCHUNK, CHUNK_PAD), jnp.int32),
            pltpu.VMEM((NCHUNK, CHUNK_PAD), jnp.int32),
            pltpu.VMEM((CHUNK_PAD, W), jnp.int32),
            pltpu.VMEM((CHUNK_PAD, W), jnp.int32),
            pltpu.VMEM((CHUNK_PAD, W), jnp.int32),
            pltpu.VMEM((CHUNK_PAD, W), jnp.int32),
            pltpu.VMEM((CHUNK_PAD, W), jnp.int32),
            pltpu.VMEM((CHUNK_PAD, W), jnp.int32),
            pltpu.SemaphoreType.DMA,
            pltpu.SemaphoreType.DMA,
            pltpu.SemaphoreType.DMA,
            pltpu.SemaphoreType.DMA,
        ],
    )(_body)
    return f(table, idx0, idx1, nmask)


def _chunk_layout(x):
    x = x.reshape(NW, NCHUNK, CHUNK)
    return jnp.pad(x, ((0, 0), (0, 0), (0, CHUNK_PAD - CHUNK)))


def kernel(input_bitarrays, output_node_input_indices, nor_mask):
    idx = output_node_input_indices.astype(jnp.int32)
    idx0 = _chunk_layout(idx[:, 0])
    idx1 = _chunk_layout(idx[:, 1])
    nmask = _chunk_layout(jnp.where(nor_mask, jnp.int32(-1), jnp.int32(0)))
    out = _nand_layer(input_bitarrays, idx0, idx1, nmask)
    return out.reshape(N_OUT, W)
